# baseline (speedup 1.0000x reference)
import jax
import jax.numpy as jnp
from jax import lax
from jax.experimental import pallas as pl
from jax.experimental.pallas import tpu as pltpu
from jax.experimental.pallas import tpu_sc as plsc

N = 10000
E = 160000
D = 256
DE = 16

NC = 2
NS = 16
NW = NC * NS

RPW = 320
TRASH_X = RPW
ACC_RX = RPW + 8
NOUT = NW * RPW
CH = 32
CAP = 128

HALF = N // 2
ROWS_E = HALF + 8
TRASH_E = HALF
SUPE = 1024

EP = 163840
SUP = 2048
SCH = SUP // 128
NBX = EP // 8
NBE = EP // 128

_MESH = plsc.VectorSubcoreMesh(core_axis_name="c", subcore_axis_name="s")

_DNUMS = lax.GatherDimensionNumbers(
    offset_dims=(), collapsed_slice_dims=(0,), start_index_map=(0,))


def _x_body(x_hbm, src_hbm, dst_hbm, out, srcv, dstv, gpk, gidx, locb,
            mbuf, okb, rows, sem, accf):
    c = lax.axis_index("c")
    s = lax.axis_index("s")
    w = c * NS + s
    nbase = w * RPW

    zero16 = jnp.zeros((16,), jnp.float32)
    iota = jnp.arange(16, dtype=jnp.int32)
    sh14 = jnp.full((16,), 14, jnp.int32)

    @pl.loop(0, (ACC_RX * D) // 16)
    def _z(r):
        accf[pl.ds(r * 16, 16)] = zero16

    pad16i = jnp.full((16,), TRASH_X * 16384, jnp.int32) + nbase
    mbuf[pl.ds(0, 16)] = pad16i
    mbuf[pl.ds(32, 16)] = pad16i
    zero16i = jnp.zeros((16,), jnp.int32)
    okb[pl.ds(0, 16)] = zero16i
    okb[pl.ds(32, 16)] = zero16i

    def accum_flush():
        for t in range(CH // 16):
            v = gpk[pl.ds(t * 16, 16)]
            locb[pl.ds(t * 16, 16)] = lax.shift_right_logical(v, sh14)
            gidx[pl.ds(t * 16, 16)] = v & 16383
        pltpu.async_copy(x_hbm.at[gidx], rows, sem).wait()
        for e in range(CH):
            if e % 16 == 0:
                lv = locb[pl.ds(e, 16)]
            row = lv[e % 16]
            for j in range(D // 16):
                plsc.addupdate(accf.at[pl.ds(row * D + j * 16, 16)],
                               rows[e, pl.ds(j * 16, 16)])

    def relocate():
        v = gpk[pl.ds(CH, 16)]
        gpk[pl.ds(0, 16)] = v

    @pl.loop(0, NBX)
    def _sweep(q):
        @pl.when(q % (SUP // 8) == 0)
        def _stage():
            gb = (q // (SUP // 8)) * SCH
            pltpu.sync_copy(dst_hbm.at[pl.ds(gb, SCH)], dstv)
            pltpu.sync_copy(src_hbm.at[pl.ds(gb, SCH)], srcv)

        m = (q // 2) % (SUP // 16)
        h = q % 2
        j = m // 8
        k = m - j * 8
        dd = dstv[j, pl.ds(k * 16, 16)]
        loc = dd - nbase
        ok = (loc >= 0) & (loc < RPW)
        okh = ok & (lax.shift_right_logical(
            iota, jnp.full((16,), 3, jnp.int32)) == h)
        val = lax.shift_left(loc, sh14) + srcv[j, pl.ds(k * 16, 16)]
        val = jnp.where(okh, val, TRASH_X * 16384 + nbase)
        mbuf[pl.ds(16, 16)] = val
        vs = mbuf[pl.ds(16 + 8 * h, 16)]
        gpk[pl.ds((q % 4) * 8, 16)] = vs

        @pl.when(q % 4 == 3)
        def _fl():
            accum_flush()

    pltpu.sync_copy(accf.at[pl.ds(0, RPW * D)],
                    out.at[pl.ds(nbase * D, RPW * D)])


_sc_xseg = pl.kernel(
    _x_body,
    out_type=jax.ShapeDtypeStruct((NOUT * D,), jnp.float32),
    mesh=_MESH,
    scratch_types=[
        pltpu.VMEM((SCH, 128), jnp.int32),
        pltpu.VMEM((SCH, 128), jnp.int32),
        pltpu.VMEM((CAP,), jnp.int32),
        pltpu.VMEM((CH,), jnp.int32),
        pltpu.VMEM((CH,), jnp.int32),
        pltpu.VMEM((48,), jnp.int32),
        pltpu.VMEM((48,), jnp.int32),
        pltpu.VMEM((CH, D), jnp.float32),
        pltpu.SemaphoreType.DMA,
        pltpu.VMEM((ACC_RX * D,), jnp.float32),
    ],
)


def _e_body(dst_hbm, ea1_hbm, ea2_hbm, out, dstv, rowb, eavf, sem, accf):
    c = lax.axis_index("c")
    s = lax.axis_index("s")
    rbase = c * HALF
    sch_e = SUPE // 128

    zero16 = jnp.zeros((16,), jnp.float32)

    @pl.loop(0, (ROWS_E * DE) // 16)
    def _z(r):
        accf[pl.ds(r * 16, 16)] = zero16

    @pl.loop(0, NBE)
    def _sweep(q):
        @pl.when(q % sch_e == 0)
        def _stage():
            g = q // sch_e
            pltpu.sync_copy(dst_hbm.at[pl.ds(g * sch_e, sch_e)], dstv)

            @pl.when(s < 8)
            def _s1():
                pltpu.sync_copy(ea1_hbm.at[pl.ds(g * SUPE * DE, SUPE * DE)],
                                eavf)

            @pl.when(s >= 8)
            def _s2():
                pltpu.sync_copy(ea2_hbm.at[pl.ds(g * SUPE * DE, SUPE * DE)],
                                eavf)

        j = q % sch_e
        for k in range(8):
            @pl.when(s % 8 == k)
            def _chunk(j=j, k=k):
                dd = dstv[j, pl.ds(k * 16, 16)]
                loc = dd - rbase
                ok = (loc >= 0) & (loc < HALF)
                rowb[pl.ds(0, 16)] = jnp.where(ok, loc, TRASH_E)
                rv = rowb[pl.ds(0, 16)]
                for lane in range(16):
                    row = rv[lane]
                    ev = eavf[pl.ds((j * 128 + k * 16 + lane) * 16, 16)]
                    plsc.addupdate(accf.at[pl.ds(row * 16, 16)], ev)

    w = c * NS + s
    pltpu.sync_copy(accf.at[pl.ds(0, ROWS_E * 16)],
                    out.at[pl.ds(w * (ROWS_E * 16), ROWS_E * 16)])


_sc_eseg = pl.kernel(
    _e_body,
    out_type=jax.ShapeDtypeStruct((NC * NS * ROWS_E * 16,), jnp.float32),
    mesh=_MESH,
    scratch_types=[
        pltpu.VMEM((SUPE // 128, 128), jnp.int32),
        pltpu.VMEM((16,), jnp.int32),
        pltpu.VMEM((SUPE * DE,), jnp.float32),
        pltpu.SemaphoreType.DMA,
        pltpu.VMEM((ROWS_E * DE,), jnp.float32),
    ],
)


def _tc_layer(x, a, egp, wr, wm, we, b):
    R = 400

    def body(x_ref, a_ref, e_ref, wr_ref, wm_ref, we_ref, b_ref, o_ref):
        acc = jnp.dot(x_ref[...], wr_ref[...], preferred_element_type=jnp.float32)
        acc = acc + jnp.dot(a_ref[...], wm_ref[...], preferred_element_type=jnp.float32)
        eg = jnp.sum(e_ref[...], axis=0)
        acc = acc + jnp.dot(eg, we_ref[...], preferred_element_type=jnp.float32)
        o_ref[...] = jnp.maximum(acc + b_ref[...], 0.0)

    return pl.pallas_call(
        body,
        grid=(N // R,),
        in_specs=[
            pl.BlockSpec((R, D), lambda i: (i, 0)),
            pl.BlockSpec((R, D), lambda i: (i, 0)),
            pl.BlockSpec((8, R, DE), lambda i: (0, i, 0)),
            pl.BlockSpec((D, D), lambda i: (0, 0)),
            pl.BlockSpec((D, D), lambda i: (0, 0)),
            pl.BlockSpec((DE, D), lambda i: (0, 0)),
            pl.BlockSpec((1, D), lambda i: (0, 0)),
        ],
        out_specs=pl.BlockSpec((R, D), lambda i: (i, 0)),
        out_shape=jax.ShapeDtypeStruct((N, D), jnp.float32),
    )(x, a, egp, wr, wm, we, b.reshape(1, D))


def kernel(flattened_data, edge_index, edge_attr, pooled_edge_attr,
           W_root1, W_msg1, W_edge1, b1,
           W_root2, W_msg2, W_edge2, b2):
    x0 = flattened_data.reshape(N, D)
    pad = EP - E
    src = jnp.concatenate([edge_index[0], jnp.zeros((pad,), jnp.int32)])
    dst = jnp.concatenate([edge_index[1], jnp.full((pad,), -1, jnp.int32)])
    src2 = src.reshape(EP // 128, 128)
    dst2 = dst.reshape(EP // 128, 128)
    ea1 = jnp.concatenate([pooled_edge_attr,
                           jnp.zeros((pad, DE), jnp.float32)]).reshape(-1)
    ea2 = jnp.concatenate([edge_attr,
                           jnp.zeros((pad, DE), jnp.float32)]).reshape(-1)

    eo = _sc_eseg(dst2, ea1, ea2)
    ep = eo.reshape(NC, NS, ROWS_E, DE)
    e1p = jnp.concatenate([ep[0, :8, :HALF, :], ep[1, :8, :HALF, :]], axis=1)
    e2p = jnp.concatenate([ep[0, 8:, :HALF, :], ep[1, 8:, :HALF, :]], axis=1)

    a1 = _sc_xseg(x0, src2, dst2).reshape(NOUT, D)
    x1 = _tc_layer(x0, a1, e1p, W_root1, W_msg1, W_edge1, b1)
    a2 = _sc_xseg(x1, src2, dst2).reshape(NOUT, D)
    x2 = _tc_layer(x1, a2, e2p, W_root2, W_msg2, W_edge2, b2)
    return x2


# speedup vs baseline: 1.7248x; 1.7248x over previous
import jax
import jax.numpy as jnp
from jax import lax
from jax.experimental import pallas as pl
from jax.experimental.pallas import tpu as pltpu
from jax.experimental.pallas import tpu_sc as plsc

N = 10000
E = 160000
D = 256
DE = 16

NC = 2
NS = 16
NW = NC * NS

RPW = 320
TRASH_X = RPW
ACC_RX = RPW + 8
NOUT = NW * RPW
CH = 32
CAP = 128

HALF = N // 2
ROWS_E = HALF + 8
TRASH_E = HALF
SUPE = 1024

EP = 163840
SUP = 2048
SCH = SUP // 128
NBX = EP // 8
NBE = EP // 128

_MESH = plsc.VectorSubcoreMesh(core_axis_name="c", subcore_axis_name="s")

_DNUMS = lax.GatherDimensionNumbers(
    offset_dims=(), collapsed_slice_dims=(0,), start_index_map=(0,))


def _x_body(x_hbm, src_hbm, dst_hbm, out, srcv, dstv, gpk, gidx, locb,
            mbuf, okb, rows, sem, accf):
    c = lax.axis_index("c")
    s = lax.axis_index("s")
    w = c * NS + s
    nbase = w * RPW

    zero16 = jnp.zeros((16,), jnp.float32)
    iota = jnp.arange(16, dtype=jnp.int32)
    sh14 = jnp.full((16,), 14, jnp.int32)

    @pl.loop(0, (ACC_RX * D) // 16)
    def _z(r):
        accf[pl.ds(r * 16, 16)] = zero16

    pad16i = jnp.full((16,), TRASH_X * 16384, jnp.int32) + nbase
    mbuf[pl.ds(0, 16)] = pad16i
    mbuf[pl.ds(32, 16)] = pad16i
    zero16i = jnp.zeros((16,), jnp.int32)
    okb[pl.ds(0, 16)] = zero16i
    okb[pl.ds(32, 16)] = zero16i

    def accum_flush():
        for t in range(CH // 16):
            v = gpk[pl.ds(t * 16, 16)]
            locb[pl.ds(t * 16, 16)] = lax.shift_right_logical(v, sh14)
            gidx[pl.ds(t * 16, 16)] = v & 16383
        pltpu.async_copy(x_hbm.at[gidx], rows, sem).wait()
        for e in range(CH):
            if e % 16 == 0:
                lv = locb[pl.ds(e, 16)]
            row = lv[e % 16]
            for j in range(D // 16):
                plsc.addupdate(accf.at[pl.ds(row * D + j * 16, 16)],
                               rows[e, pl.ds(j * 16, 16)])

    def relocate():
        v = gpk[pl.ds(CH, 16)]
        gpk[pl.ds(0, 16)] = v

    @pl.loop(0, NBX)
    def _sweep(q):
        @pl.when(q % (SUP // 8) == 0)
        def _stage():
            gb = (q // (SUP // 8)) * SCH
            pltpu.sync_copy(dst_hbm.at[pl.ds(gb, SCH)], dstv)
            pltpu.sync_copy(src_hbm.at[pl.ds(gb, SCH)], srcv)

        m = (q // 2) % (SUP // 16)
        h = q % 2
        j = m // 8
        k = m - j * 8
        dd = dstv[j, pl.ds(k * 16, 16)]
        loc = dd - nbase
        ok = (loc >= 0) & (loc < RPW)
        okh = ok & (lax.shift_right_logical(
            iota, jnp.full((16,), 3, jnp.int32)) == h)
        val = lax.shift_left(loc, sh14) + srcv[j, pl.ds(k * 16, 16)]
        spread = iota & 7
        val = jnp.where(okh,
                        val,
                        lax.shift_left(TRASH_X + spread, sh14)
                        + (nbase + spread))
        mbuf[pl.ds(16, 16)] = val
        vs = mbuf[pl.ds(16 + 8 * h, 16)]
        gpk[pl.ds((q % 4) * 8, 16)] = vs

        @pl.when(q % 4 == 3)
        def _fl():
            accum_flush()

    pltpu.sync_copy(accf.at[pl.ds(0, RPW * D)],
                    out.at[pl.ds(nbase * D, RPW * D)])


_sc_xseg = pl.kernel(
    _x_body,
    out_type=jax.ShapeDtypeStruct((NOUT * D,), jnp.float32),
    mesh=_MESH,
    scratch_types=[
        pltpu.VMEM((SCH, 128), jnp.int32),
        pltpu.VMEM((SCH, 128), jnp.int32),
        pltpu.VMEM((CAP,), jnp.int32),
        pltpu.VMEM((CH,), jnp.int32),
        pltpu.VMEM((CH,), jnp.int32),
        pltpu.VMEM((48,), jnp.int32),
        pltpu.VMEM((48,), jnp.int32),
        pltpu.VMEM((CH, D), jnp.float32),
        pltpu.SemaphoreType.DMA,
        pltpu.VMEM((ACC_RX * D,), jnp.float32),
    ],
)


def _e_body(dst_hbm, ea1_hbm, ea2_hbm, out, dstv, rowb, eavf, sem, accf):
    c = lax.axis_index("c")
    s = lax.axis_index("s")
    rbase = c * HALF
    sch_e = SUPE // 128

    zero16 = jnp.zeros((16,), jnp.float32)

    @pl.loop(0, (ROWS_E * DE) // 16)
    def _z(r):
        accf[pl.ds(r * 16, 16)] = zero16

    @pl.loop(0, NBE)
    def _sweep(q):
        @pl.when(q % sch_e == 0)
        def _stage():
            g = q // sch_e
            pltpu.sync_copy(dst_hbm.at[pl.ds(g * sch_e, sch_e)], dstv)

            @pl.when(s < 8)
            def _s1():
                pltpu.sync_copy(ea1_hbm.at[pl.ds(g * SUPE * DE, SUPE * DE)],
                                eavf)

            @pl.when(s >= 8)
            def _s2():
                pltpu.sync_copy(ea2_hbm.at[pl.ds(g * SUPE * DE, SUPE * DE)],
                                eavf)

        j = q % sch_e
        for k in range(8):
            @pl.when(s % 8 == k)
            def _chunk(j=j, k=k):
                dd = dstv[j, pl.ds(k * 16, 16)]
                loc = dd - rbase
                ok = (loc >= 0) & (loc < HALF)
                rowb[pl.ds(0, 16)] = jnp.where(
                    ok, loc, TRASH_E + (jnp.arange(16, dtype=jnp.int32) & 7))
                rv = rowb[pl.ds(0, 16)]
                for lane in range(16):
                    row = rv[lane]
                    ev = eavf[pl.ds((j * 128 + k * 16 + lane) * 16, 16)]
                    plsc.addupdate(accf.at[pl.ds(row * 16, 16)], ev)

    w = c * NS + s
    pltpu.sync_copy(accf.at[pl.ds(0, ROWS_E * 16)],
                    out.at[pl.ds(w * (ROWS_E * 16), ROWS_E * 16)])


_sc_eseg = pl.kernel(
    _e_body,
    out_type=jax.ShapeDtypeStruct((NC * NS * ROWS_E * 16,), jnp.float32),
    mesh=_MESH,
    scratch_types=[
        pltpu.VMEM((SUPE // 128, 128), jnp.int32),
        pltpu.VMEM((16,), jnp.int32),
        pltpu.VMEM((SUPE * DE,), jnp.float32),
        pltpu.SemaphoreType.DMA,
        pltpu.VMEM((ROWS_E * DE,), jnp.float32),
    ],
)


def _tc_layer(x, a, egp, wr, wm, we, b):
    R = 400

    def body(x_ref, a_ref, e_ref, wr_ref, wm_ref, we_ref, b_ref, o_ref):
        acc = jnp.dot(x_ref[...], wr_ref[...], preferred_element_type=jnp.float32)
        acc = acc + jnp.dot(a_ref[...], wm_ref[...], preferred_element_type=jnp.float32)
        eg = jnp.sum(e_ref[...], axis=0)
        acc = acc + jnp.dot(eg, we_ref[...], preferred_element_type=jnp.float32)
        o_ref[...] = jnp.maximum(acc + b_ref[...], 0.0)

    return pl.pallas_call(
        body,
        grid=(N // R,),
        in_specs=[
            pl.BlockSpec((R, D), lambda i: (i, 0)),
            pl.BlockSpec((R, D), lambda i: (i, 0)),
            pl.BlockSpec((8, R, DE), lambda i: (0, i, 0)),
            pl.BlockSpec((D, D), lambda i: (0, 0)),
            pl.BlockSpec((D, D), lambda i: (0, 0)),
            pl.BlockSpec((DE, D), lambda i: (0, 0)),
            pl.BlockSpec((1, D), lambda i: (0, 0)),
        ],
        out_specs=pl.BlockSpec((R, D), lambda i: (i, 0)),
        out_shape=jax.ShapeDtypeStruct((N, D), jnp.float32),
    )(x, a, egp, wr, wm, we, b.reshape(1, D))


def kernel(flattened_data, edge_index, edge_attr, pooled_edge_attr,
           W_root1, W_msg1, W_edge1, b1,
           W_root2, W_msg2, W_edge2, b2):
    x0 = flattened_data.reshape(N, D)
    pad = EP - E
    src = jnp.concatenate([edge_index[0], jnp.zeros((pad,), jnp.int32)])
    dst = jnp.concatenate([edge_index[1], jnp.full((pad,), -1, jnp.int32)])
    src2 = src.reshape(EP // 128, 128)
    dst2 = dst.reshape(EP // 128, 128)
    ea1 = jnp.concatenate([pooled_edge_attr,
                           jnp.zeros((pad, DE), jnp.float32)]).reshape(-1)
    ea2 = jnp.concatenate([edge_attr,
                           jnp.zeros((pad, DE), jnp.float32)]).reshape(-1)

    eo = _sc_eseg(dst2, ea1, ea2)
    ep = eo.reshape(NC, NS, ROWS_E, DE)
    e1p = jnp.concatenate([ep[0, :8, :HALF, :], ep[1, :8, :HALF, :]], axis=1)
    e2p = jnp.concatenate([ep[0, 8:, :HALF, :], ep[1, 8:, :HALF, :]], axis=1)

    a1 = _sc_xseg(x0, src2, dst2).reshape(NOUT, D)
    x1 = _tc_layer(x0, a1, e1p, W_root1, W_msg1, W_edge1, b1)
    a2 = _sc_xseg(x1, src2, dst2).reshape(NOUT, D)
    x2 = _tc_layer(x1, a2, e2p, W_root2, W_msg2, W_edge2, b2)
    return x2
